# Initial kernel scaffold; baseline (speedup 1.0000x reference)
import jax
import jax.numpy as jnp
from jax.experimental import pallas as pl


def kernel(inp, weights, tau_syn_weights, indices):
    raise NotImplementedError("write your pallas kernel here")


# speedup vs baseline: 2.8264x; 2.8264x over previous
import functools

import jax
import jax.numpy as jnp
from jax import lax
from jax.experimental import pallas as pl
from jax.experimental.pallas import tpu as pltpu
from jax.experimental.pallas import tpu_sc as plsc

NOUT = 16384
NB = 5
LANES = 16
NC = 2
NS = 16
NW = NC * NS
RPW = NOUT // NW
PTR_SLICE = 528


def _sc_spmm(xT, colsp, wrp, ptrp, bt):
    acc_len = NB * bt
    mesh = plsc.VectorSubcoreMesh(core_axis_name="c", subcore_axis_name="s")

    @functools.partial(
        pl.kernel,
        out_type=jax.ShapeDtypeStruct((NOUT, acc_len), jnp.float32),
        mesh=mesh,
        scratch_types=[
            pltpu.VMEM((PTR_SLICE,), jnp.int32),
            pltpu.VMEM((LANES,), jnp.int32),
            pltpu.VMEM((LANES * NB,), jnp.float32),
            pltpu.VMEM((LANES, bt), jnp.float32),
            pltpu.VMEM((acc_len,), jnp.float32),
            pltpu.SemaphoreType.DMA,
        ],
    )
    def spmm(xT_hbm, cols_hbm, wr_hbm, ptr_hbm, out_hbm,
             ptrv, colv, wrv, gbuf, acc, sem):
        wid = lax.axis_index("s") * NC + lax.axis_index("c")
        row0 = wid * RPW
        pltpu.sync_copy(ptr_hbm.at[pl.ds(row0, PTR_SLICE)], ptrv)

        @pl.loop(0, RPW)
        def _row(o):
            pv = ptrv[pl.ds(o, LANES)]
            start = pv[0]
            end = pv[1]

            @pl.loop(0, acc_len, step=LANES)
            def _zero(i):
                acc[pl.ds(i, LANES)] = jnp.zeros((LANES,), jnp.float32)

            astart = (start // LANES) * LANES
            nch = jnp.where(end > start, (end - astart + LANES - 1) // LANES, 0)

            def chunk_body(j, carry):
                kbase = astart + j * LANES
                pltpu.sync_copy(cols_hbm.at[pl.ds(kbase, LANES)], colv)
                pltpu.sync_copy(wr_hbm.at[pl.ds(kbase * NB, LANES * NB)], wrv)
                pltpu.async_copy(xT_hbm.at[colv], gbuf, sem).wait()
                wv = [wrv[pl.ds(t * LANES, LANES)] for t in range(NB)]
                ws = []
                for kk in range(LANES):
                    kg = kbase + kk
                    valid = jnp.logical_and(kg >= start, kg < end)
                    for r in range(NB):
                        flat = kk * NB + r
                        ws.append(jnp.where(valid,
                                            wv[flat // LANES][flat % LANES],
                                            jnp.float32(0.0)))

                @pl.loop(0, bt, step=4 * LANES)
                def _sg(sgb):
                    accs = [acc[pl.ds(r * bt + sgb + t * LANES, LANES)]
                            for r in range(NB) for t in range(4)]
                    for kk in range(LANES):
                        xv = [gbuf[kk, pl.ds(sgb + t * LANES, LANES)]
                              for t in range(4)]
                        for r in range(NB):
                            w = ws[kk * NB + r]
                            for t in range(4):
                                accs[r * 4 + t] = accs[r * 4 + t] + w * xv[t]
                    for r in range(NB):
                        for t in range(4):
                            acc[pl.ds(r * bt + sgb + t * LANES, LANES)] = \
                                accs[r * 4 + t]

                return carry

            lax.fori_loop(0, nch, chunk_body, 0)
            pltpu.sync_copy(acc, out_hbm.at[row0 + o])

    return spmm(xT, colsp, wrp, ptrp)


def kernel(inp, weights, tau_syn_weights, indices):
    b, t, n_in = inp.shape
    bt = b * t
    x = inp.reshape(bt, n_in)
    xT = x.T
    rows = indices[:, 0]
    cols = indices[:, 1]
    nnz = rows.shape[0]
    nnz_pad = ((nnz + LANES - 1) // LANES) * LANES
    colsp = jnp.zeros((nnz_pad,), jnp.int32).at[:nnz].set(cols)
    wr = weights[:, None] * tau_syn_weights
    wrp = jnp.zeros((nnz_pad, NB), jnp.float32).at[:nnz].set(wr).reshape(-1)
    ptr = jnp.searchsorted(rows, jnp.arange(NOUT + 1)).astype(jnp.int32)
    ptr_len = (NW - 1) * RPW + PTR_SLICE
    ptrp = jnp.full((ptr_len,), nnz, jnp.int32).at[:NOUT + 1].set(ptr)
    tmp = _sc_spmm(xT, colsp, wrp, ptrp, bt)
    out = tmp.reshape(NOUT, NB, bt).transpose(2, 0, 1)
    return out.reshape(b, t, NOUT * NB)


# speedup vs baseline: 4.4948x; 1.5903x over previous
import functools

import jax
import jax.numpy as jnp
from jax import lax
from jax.experimental import pallas as pl
from jax.experimental.pallas import tpu as pltpu
from jax.experimental.pallas import tpu_sc as plsc

NOUT = 16384
NB = 5
LANES = 16
NC = 2
NS = 16
NW = NC * NS
RPW = NOUT // NW
PTR_SLICE = 528
BLK = 1024
STG = BLK + 128


def _sc_spmm(xT, colsp, wrp, ptrp, bt):
    acc_len = NB * bt
    mesh = plsc.VectorSubcoreMesh(core_axis_name="c", subcore_axis_name="s")

    @functools.partial(
        pl.kernel,
        out_type=jax.ShapeDtypeStruct((NOUT, acc_len), jnp.float32),
        mesh=mesh,
        scratch_types=[
            pltpu.VMEM((PTR_SLICE,), jnp.int32),
            pltpu.VMEM((LANES,), jnp.int32),
            pltpu.VMEM((LANES,), jnp.int32),
            pltpu.VMEM((STG,), jnp.int32),
            pltpu.VMEM((NB, STG), jnp.float32),
            pltpu.VMEM((LANES, bt), jnp.float32),
            pltpu.VMEM((LANES, bt), jnp.float32),
            pltpu.VMEM((NB * LANES,), jnp.float32),
            pltpu.VMEM((NB * LANES,), jnp.float32),
            pltpu.VMEM((acc_len,), jnp.float32),
            pltpu.SemaphoreType.DMA,
            pltpu.SemaphoreType.DMA,
            pltpu.SemaphoreType.DMA,
        ],
    )
    def spmm(xT_hbm, cols_hbm, wr_hbm, ptr_hbm, out_hbm,
             ptrv, colvA, colvB, colstg, wrstg, gbufA, gbufB,
             wbufA, wbufB, acc, semA, semB, semO):
        wid = lax.axis_index("s") * NC + lax.axis_index("c")
        row0 = wid * RPW
        pltpu.sync_copy(ptr_hbm.at[pl.ds(row0, PTR_SLICE)], ptrv)
        zvec = jnp.zeros((LANES,), jnp.float32)

        def issue(kbase, first, end, cached_b, colv, gbuf, wbuf, sem):
            b0 = kbase // BLK

            def restage(cb):
                pltpu.sync_copy(cols_hbm.at[pl.ds(b0 * BLK, STG)], colstg)
                pltpu.sync_copy(wr_hbm.at[:, pl.ds(b0 * BLK, STG)], wrstg)
                return b0

            cached_b = lax.cond(b0 != cached_b, restage, lambda cb: cb,
                                cached_b)
            koff = kbase - b0 * BLK
            colv[...] = colstg[pl.ds(koff, LANES)]
            pltpu.async_copy(xT_hbm.at[colv], gbuf, sem)
            kg = kbase + lax.broadcasted_iota(jnp.int32, (LANES,), 0)
            msk = jnp.logical_and(kg >= first, kg < end)
            for r in range(NB):
                wbuf[pl.ds(r * LANES, LANES)] = jnp.where(
                    msk, wrstg[r, pl.ds(koff, LANES)], zvec)
            return cached_b

        def wait_gather(colv, gbuf, sem):
            pltpu.make_async_copy(xT_hbm.at[colv], gbuf, sem).wait()

        def compute(gbuf, wbuf):
            @pl.loop(0, bt, step=4 * LANES)
            def _sg(sgb):
                wvm = [wbuf[pl.ds(r * LANES, LANES)] for r in range(NB)]
                accs = [acc[pl.ds(r * bt + sgb + t * LANES, LANES)]
                        for r in range(NB) for t in range(4)]
                for kk in range(LANES):
                    xv = [gbuf[kk, pl.ds(sgb + t * LANES, LANES)]
                          for t in range(4)]
                    for r in range(NB):
                        w = wvm[r][kk]
                        for t in range(4):
                            accs[r * 4 + t] = accs[r * 4 + t] + w * xv[t]
                for r in range(NB):
                    for t in range(4):
                        acc[pl.ds(r * bt + sgb + t * LANES, LANES)] = \
                            accs[r * 4 + t]

        def row_body(o, cached_b):
            pv = ptrv[pl.ds(o, LANES)]
            start = pv[0]
            end = pv[1]
            astart = (start // LANES) * LANES
            nch = jnp.where(end > start,
                            (end - astart + LANES - 1) // LANES, 0)

            cached_b = lax.cond(
                nch > 0,
                lambda cb: issue(astart, start, end, cb, colvA, gbufA,
                                 wbufA, semA),
                lambda cb: cb,
                cached_b)

            @pl.when(o > 0)
            def _():
                pltpu.make_async_copy(acc, out_hbm.at[row0], semO).wait()

            @pl.loop(0, acc_len, step=LANES)
            def _zero(i):
                acc[pl.ds(i, LANES)] = zvec

            def pair_body(p, cached_b):
                j1 = 2 * p + 1

                cached_b = lax.cond(
                    j1 < nch,
                    lambda cb: issue(astart + j1 * LANES, start, end, cb,
                                     colvB, gbufB, wbufB, semB),
                    lambda cb: cb,
                    cached_b)

                wait_gather(colvA, gbufA, semA)
                compute(gbufA, wbufA)

                cached_b = lax.cond(
                    j1 + 1 < nch,
                    lambda cb: issue(astart + (j1 + 1) * LANES, start, end,
                                     cb, colvA, gbufA, wbufA, semA),
                    lambda cb: cb,
                    cached_b)

                @pl.when(j1 < nch)
                def _():
                    wait_gather(colvB, gbufB, semB)
                    compute(gbufB, wbufB)

                return cached_b

            cached_b = lax.fori_loop(0, (nch + 1) // 2, pair_body, cached_b)
            pltpu.async_copy(acc, out_hbm.at[row0 + o], semO)
            return cached_b

        lax.fori_loop(0, RPW, row_body, jnp.int32(-1))
        pltpu.make_async_copy(acc, out_hbm.at[row0], semO).wait()

    return spmm(xT, colsp, wrp, ptrp)


def kernel(inp, weights, tau_syn_weights, indices):
    b, t, n_in = inp.shape
    bt = b * t
    x = inp.reshape(bt, n_in)
    xT = x.T
    rows = indices[:, 0]
    cols = indices[:, 1]
    nnz = rows.shape[0]
    nnz_stg = ((nnz + BLK - 1) // BLK) * BLK + 128
    colsp = jnp.zeros((nnz_stg,), jnp.int32).at[:nnz].set(cols)
    wr = weights[:, None] * tau_syn_weights
    wrp = jnp.zeros((nnz_stg, NB), jnp.float32).at[:nnz].set(wr).T
    ptr = jnp.searchsorted(rows, jnp.arange(NOUT + 1)).astype(jnp.int32)
    ptr_len = (NW - 1) * RPW + PTR_SLICE
    ptrp = jnp.full((ptr_len,), nnz, jnp.int32).at[:NOUT + 1].set(ptr)
    tmp = _sc_spmm(xT, colsp, wrp, ptrp, bt)
    out = tmp.reshape(NOUT, NB, bt).transpose(2, 0, 1)
    return out.reshape(b, t, NOUT * NB)


# speedup vs baseline: 4.9622x; 1.1040x over previous
import dataclasses
import functools

import jax
import jax.numpy as jnp
from jax import lax
from jax.experimental import pallas as pl
from jax.experimental.pallas import tpu as pltpu
from jax.experimental.pallas import tpu_sc as plsc

NOUT = 16384
NB = 5
LANES = 16
NC = 2
NS = 16
NW = NC * NS
RPW = NOUT // NW
PTR_SLICE = 528
BLK = 1024
STG = BLK + 128


def _sc_spmm(xT, colsp, wrp, ptrp, bt):
    acc_len = NB * bt
    mesh = plsc.VectorSubcoreMesh(core_axis_name="c", subcore_axis_name="s")
    cp = pltpu.CompilerParams()
    if "needs_layout_passes" in pltpu.CompilerParams.__dataclass_fields__:
        cp = dataclasses.replace(cp, needs_layout_passes=False)

    @functools.partial(
        pl.kernel,
        out_type=jax.ShapeDtypeStruct((NOUT, acc_len), jnp.float32),
        mesh=mesh,
        compiler_params=cp,
        scratch_types=[
            pltpu.VMEM((PTR_SLICE,), jnp.int32),
            pltpu.VMEM((LANES,), jnp.int32),
            pltpu.VMEM((LANES,), jnp.int32),
            pltpu.VMEM((STG,), jnp.int32),
            pltpu.VMEM((NB, STG), jnp.float32),
            pltpu.VMEM((LANES, bt), jnp.float32),
            pltpu.VMEM((LANES, bt), jnp.float32),
            pltpu.VMEM((NB * LANES,), jnp.float32),
            pltpu.VMEM((NB * LANES,), jnp.float32),
            pltpu.VMEM((acc_len,), jnp.float32),
            pltpu.SemaphoreType.DMA,
            pltpu.SemaphoreType.DMA,
            pltpu.SemaphoreType.DMA,
        ],
    )
    def spmm(xT_hbm, cols_hbm, wr_hbm, ptr_hbm, out_hbm,
             ptrv, colvA, colvB, colstg, wrstg, gbufA, gbufB,
             wbufA, wbufB, acc, semA, semB, semO):
        wid = lax.axis_index("s") * NC + lax.axis_index("c")
        row0 = wid * RPW
        pltpu.sync_copy(ptr_hbm.at[pl.ds(row0, PTR_SLICE)], ptrv)
        zvec = jnp.zeros((LANES,), jnp.float32)

        def issue(kbase, end, cached_b, colv, gbuf, wbuf, sem):
            b0 = kbase // BLK

            def restage(cb):
                pltpu.sync_copy(cols_hbm.at[pl.ds(b0 * BLK, STG)], colstg)
                pltpu.sync_copy(wr_hbm.at[:, pl.ds(b0 * BLK, STG)], wrstg)
                return b0

            cached_b = lax.cond(b0 != cached_b, restage, lambda cb: cb,
                                cached_b)
            lane = lax.broadcasted_iota(jnp.int32, (LANES,), 0)
            idx = (kbase - b0 * BLK) + lane
            colv[...] = plsc.load_gather(colstg, [idx])
            pltpu.async_copy(xT_hbm.at[colv], gbuf, sem)
            msk = (kbase + lane) < end
            for r in range(NB):
                wbuf[pl.ds(r * LANES, LANES)] = jnp.where(
                    msk,
                    plsc.load_gather(
                        wrstg, [jnp.full((LANES,), r, jnp.int32), idx]),
                    zvec)
            return cached_b

        def wait_gather(colv, gbuf, sem):
            pltpu.make_async_copy(xT_hbm.at[colv], gbuf, sem).wait()

        def compute(gbuf, wbuf):
            @pl.loop(0, bt, step=4 * LANES)
            def _sg(sgb):
                wvm = [wbuf[pl.ds(r * LANES, LANES)] for r in range(NB)]
                accs = [acc[pl.ds(r * bt + sgb + t * LANES, LANES)]
                        for r in range(NB) for t in range(4)]
                for kk in range(LANES):
                    xv = [gbuf[kk, pl.ds(sgb + t * LANES, LANES)]
                          for t in range(4)]
                    for r in range(NB):
                        w = wvm[r][kk]
                        for t in range(4):
                            accs[r * 4 + t] = accs[r * 4 + t] + w * xv[t]
                for r in range(NB):
                    for t in range(4):
                        acc[pl.ds(r * bt + sgb + t * LANES, LANES)] = \
                            accs[r * 4 + t]

        def row_body(o, cached_b):
            pv = ptrv[pl.ds(o, LANES)]
            start = pv[0]
            end = pv[1]
            nch = jnp.where(end > start,
                            (end - start + LANES - 1) // LANES, 0)

            cached_b = lax.cond(
                nch > 0,
                lambda cb: issue(start, end, cb, colvA, gbufA,
                                 wbufA, semA),
                lambda cb: cb,
                cached_b)

            @pl.when(o > 0)
            def _():
                pltpu.make_async_copy(acc, out_hbm.at[row0], semO).wait()

            @pl.loop(0, acc_len, step=LANES)
            def _zero(i):
                acc[pl.ds(i, LANES)] = zvec

            def pair_body(p, cached_b):
                j1 = 2 * p + 1

                cached_b = lax.cond(
                    j1 < nch,
                    lambda cb: issue(start + j1 * LANES, end, cb,
                                     colvB, gbufB, wbufB, semB),
                    lambda cb: cb,
                    cached_b)

                wait_gather(colvA, gbufA, semA)
                compute(gbufA, wbufA)

                cached_b = lax.cond(
                    j1 + 1 < nch,
                    lambda cb: issue(start + (j1 + 1) * LANES, end,
                                     cb, colvA, gbufA, wbufA, semA),
                    lambda cb: cb,
                    cached_b)

                @pl.when(j1 < nch)
                def _():
                    wait_gather(colvB, gbufB, semB)
                    compute(gbufB, wbufB)

                return cached_b

            cached_b = lax.fori_loop(0, (nch + 1) // 2, pair_body, cached_b)
            pltpu.async_copy(acc, out_hbm.at[row0 + o], semO)
            return cached_b

        lax.fori_loop(0, RPW, row_body, jnp.int32(-1))
        pltpu.make_async_copy(acc, out_hbm.at[row0], semO).wait()

    return spmm(xT, colsp, wrp, ptrp)


def kernel(inp, weights, tau_syn_weights, indices):
    b, t, n_in = inp.shape
    bt = b * t
    x = inp.reshape(bt, n_in)
    xT = x.T
    rows = indices[:, 0]
    cols = indices[:, 1]
    nnz = rows.shape[0]
    nnz_stg = ((nnz + BLK - 1) // BLK) * BLK + 128
    colsp = jnp.zeros((nnz_stg,), jnp.int32).at[:nnz].set(cols)
    wr = weights[:, None] * tau_syn_weights
    wrp = jnp.zeros((nnz_stg, NB), jnp.float32).at[:nnz].set(wr).T
    ptr = jnp.searchsorted(rows, jnp.arange(NOUT + 1)).astype(jnp.int32)
    ptr_len = (NW - 1) * RPW + PTR_SLICE
    ptrp = jnp.full((ptr_len,), nnz, jnp.int32).at[:NOUT + 1].set(ptr)
    tmp = _sc_spmm(xT, colsp, wrp, ptrp, bt)
    out = tmp.reshape(NOUT, NB, bt).transpose(2, 0, 1)
    return out.reshape(b, t, NOUT * NB)


# speedup vs baseline: 5.0876x; 1.0253x over previous
import dataclasses
import functools

import jax
import jax.numpy as jnp
from jax import lax
from jax.experimental import pallas as pl
from jax.experimental.pallas import tpu as pltpu
from jax.experimental.pallas import tpu_sc as plsc

NOUT = 16384
NB = 5
LANES = 16
NC = 2
NS = 16
NW = NC * NS
RPW = NOUT // NW
PTR_SLICE = 528
BLK = 1024
STG = BLK + 128


def _sc_spmm(xT, colsp, wrp, ptrp, bt):
    acc_len = NB * bt
    mesh = plsc.VectorSubcoreMesh(core_axis_name="c", subcore_axis_name="s")
    cp = pltpu.CompilerParams()
    if "needs_layout_passes" in pltpu.CompilerParams.__dataclass_fields__:
        cp = dataclasses.replace(cp, needs_layout_passes=False)

    @functools.partial(
        pl.kernel,
        out_type=jax.ShapeDtypeStruct((NOUT, acc_len), jnp.float32),
        mesh=mesh,
        compiler_params=cp,
        scratch_types=[
            pltpu.VMEM((PTR_SLICE,), jnp.int32),
            pltpu.VMEM((LANES,), jnp.int32),
            pltpu.VMEM((LANES,), jnp.int32),
            pltpu.VMEM((STG,), jnp.int32),
            pltpu.VMEM((NB, STG), jnp.float32),
            pltpu.VMEM((LANES, bt), jnp.float32),
            pltpu.VMEM((LANES, bt), jnp.float32),
            pltpu.VMEM((NB * LANES,), jnp.float32),
            pltpu.VMEM((NB * LANES,), jnp.float32),
            pltpu.VMEM((acc_len,), jnp.float32),
            pltpu.SemaphoreType.DMA,
            pltpu.SemaphoreType.DMA,
            pltpu.SemaphoreType.DMA,
        ],
    )
    def spmm(xT_hbm, cols_hbm, wr_hbm, ptr_hbm, out_hbm,
             ptrv, colvA, colvB, colstg, wrstg, gbufA, gbufB,
             wbufA, wbufB, acc, semA, semB, semO):
        wid = lax.axis_index("s") * NC + lax.axis_index("c")
        row0 = wid * RPW
        pltpu.sync_copy(ptr_hbm.at[pl.ds(row0, PTR_SLICE)], ptrv)
        zvec = jnp.zeros((LANES,), jnp.float32)

        def issue(kbase, end, cached_b, colv, gbuf, wbuf, sem):
            b0 = kbase // BLK

            def restage(cb):
                pltpu.sync_copy(cols_hbm.at[pl.ds(b0 * BLK, STG)], colstg)
                pltpu.sync_copy(wr_hbm.at[:, pl.ds(b0 * BLK, STG)], wrstg)
                return b0

            cached_b = lax.cond(b0 != cached_b, restage, lambda cb: cb,
                                cached_b)
            lane = lax.broadcasted_iota(jnp.int32, (LANES,), 0)
            idx = (kbase - b0 * BLK) + lane
            colv[...] = plsc.load_gather(colstg, [idx])
            pltpu.async_copy(xT_hbm.at[colv], gbuf, sem)
            msk = (kbase + lane) < end
            for r in range(NB):
                wbuf[pl.ds(r * LANES, LANES)] = jnp.where(
                    msk,
                    plsc.load_gather(
                        wrstg, [jnp.full((LANES,), r, jnp.int32), idx]),
                    zvec)
            return cached_b

        def wait_gather(colv, gbuf, sem):
            pltpu.make_async_copy(xT_hbm.at[colv], gbuf, sem).wait()

        def compute(gbuf, wbuf, overwrite):
            @pl.loop(0, bt, step=4 * LANES)
            def _sg(sgb):
                wvm = [wbuf[pl.ds(r * LANES, LANES)] for r in range(NB)]
                xv = [gbuf[0, pl.ds(sgb + t * LANES, LANES)]
                      for t in range(4)]
                if overwrite:
                    accs = [wvm[r][0] * xv[t]
                            for r in range(NB) for t in range(4)]
                else:
                    accs = [acc[pl.ds(r * bt + sgb + t * LANES, LANES)]
                            for r in range(NB) for t in range(4)]
                    for r in range(NB):
                        w = wvm[r][0]
                        for t in range(4):
                            accs[r * 4 + t] = accs[r * 4 + t] + w * xv[t]
                for kk in range(1, LANES):
                    xv = [gbuf[kk, pl.ds(sgb + t * LANES, LANES)]
                          for t in range(4)]
                    for r in range(NB):
                        w = wvm[r][kk]
                        for t in range(4):
                            accs[r * 4 + t] = accs[r * 4 + t] + w * xv[t]
                for r in range(NB):
                    for t in range(4):
                        acc[pl.ds(r * bt + sgb + t * LANES, LANES)] = \
                            accs[r * 4 + t]

        def row_body(o, cached_b):
            pv = ptrv[pl.ds(o, LANES)]
            start = pv[0]
            end = pv[1]
            nch = jnp.where(end > start,
                            (end - start + LANES - 1) // LANES, 0)

            cached_b = lax.cond(
                nch > 0,
                lambda cb: issue(start, end, cb, colvA, gbufA,
                                 wbufA, semA),
                lambda cb: cb,
                cached_b)

            @pl.when(o > 0)
            def _():
                pltpu.make_async_copy(acc, out_hbm.at[row0], semO).wait()

            @pl.when(nch == 0)
            def _():
                @pl.loop(0, acc_len, step=LANES)
                def _zero(i):
                    acc[pl.ds(i, LANES)] = zvec

            def pair_body(p, cached_b):
                j1 = 2 * p + 1

                cached_b = lax.cond(
                    j1 < nch,
                    lambda cb: issue(start + j1 * LANES, end, cb,
                                     colvB, gbufB, wbufB, semB),
                    lambda cb: cb,
                    cached_b)

                wait_gather(colvA, gbufA, semA)
                lax.cond(p == 0,
                         lambda: compute(gbufA, wbufA, True),
                         lambda: compute(gbufA, wbufA, False))

                cached_b = lax.cond(
                    j1 + 1 < nch,
                    lambda cb: issue(start + (j1 + 1) * LANES, end,
                                     cb, colvA, gbufA, wbufA, semA),
                    lambda cb: cb,
                    cached_b)

                @pl.when(j1 < nch)
                def _():
                    wait_gather(colvB, gbufB, semB)
                    compute(gbufB, wbufB, False)

                return cached_b

            cached_b = lax.fori_loop(0, (nch + 1) // 2, pair_body, cached_b)
            pltpu.async_copy(acc, out_hbm.at[row0 + o], semO)
            return cached_b

        lax.fori_loop(0, RPW, row_body, jnp.int32(-1))
        pltpu.make_async_copy(acc, out_hbm.at[row0], semO).wait()

    return spmm(xT, colsp, wrp, ptrp)


def kernel(inp, weights, tau_syn_weights, indices):
    b, t, n_in = inp.shape
    bt = b * t
    x = inp.reshape(bt, n_in)
    xT = x.T
    rows = indices[:, 0]
    cols = indices[:, 1]
    nnz = rows.shape[0]
    nnz_stg = ((nnz + BLK - 1) // BLK) * BLK + 128
    colsp = jnp.zeros((nnz_stg,), jnp.int32).at[:nnz].set(cols)
    wr = weights[:, None] * tau_syn_weights
    wrp = jnp.zeros((nnz_stg, NB), jnp.float32).at[:nnz].set(wr).T
    ptr = jnp.searchsorted(rows, jnp.arange(NOUT + 1)).astype(jnp.int32)
    ptr_len = (NW - 1) * RPW + PTR_SLICE
    ptrp = jnp.full((ptr_len,), nnz, jnp.int32).at[:NOUT + 1].set(ptr)
    tmp = _sc_spmm(xT, colsp, wrp, ptrp, bt)
    out = tmp.reshape(NOUT, NB, bt).transpose(2, 0, 1)
    return out.reshape(b, t, NOUT * NB)
